# Initial kernel scaffold; baseline (speedup 1.0000x reference)
import jax
import jax.numpy as jnp
from jax.experimental import pallas as pl


def kernel(x, pos, batch, params):
    raise NotImplementedError("write your pallas kernel here")


# speedup vs baseline: 8.5409x; 8.5409x over previous
import functools

import jax
import jax.numpy as jnp
from jax import lax
from jax.experimental import pallas as pl
from jax.experimental.pallas import tpu as pltpu
from jax.experimental.pallas import tpu_sc as plsc

_PC = pl.pallas_call

N = 8192
N1, N1P = 1639, 1664
N2, N2P = 410, 512
K = 64
R1SQ = 0.2 * 0.2
R2SQ = 0.4 * 0.4
BIGI = 2**30
HIGHEST = jax.lax.Precision.HIGHEST
DEF = jax.lax.Precision.DEFAULT


def _ids2d(rows, cols):
    return (lax.broadcasted_iota(jnp.int32, (rows, cols), 0) * cols
            + lax.broadcasted_iota(jnp.int32, (rows, cols), 1))


def _fps_body(n_valid, n_samples, q_rows, pos_ref, q_ref):
    rows = pos_ref.shape[1]
    px, py, pz = pos_ref[0], pos_ref[1], pos_ref[2]
    ids = _ids2d(rows, 128)
    col = lax.broadcasted_iota(jnp.int32, (1, 128), 1)

    lx, ly, lz = px[0, 0], py[0, 0], pz[0, 0]
    min_d = jnp.where(ids < n_valid, jnp.inf, -1.0)

    for r in range(q_rows):
        gbase = r * 128
        pad = jnp.where(col + gbase >= n_samples, 4.0, 0.0)
        qxr, qyr, qzr = pad, pad, pad
        if r == 0:
            qxr = jnp.where(col == 0, lx, qxr)
            qyr = jnp.where(col == 0, ly, qyr)
            qzr = jnp.where(col == 0, lz, qzr)

        def body(i, c, gbase=gbase):
            min_d, lx, ly, lz, qxr, qyr, qzr = c
            d = (px - lx) ** 2 + (py - ly) ** 2 + (pz - lz) ** 2
            min_d = jnp.minimum(min_d, d)
            m = jnp.max(min_d)
            nxt = jnp.min(jnp.where(min_d == m, ids, BIGI))
            rid = nxt // 128
            selc = col == (nxt - rid * 128)
            pxr = pos_ref[0, pl.ds(rid, 1), :]
            pyr = pos_ref[1, pl.ds(rid, 1), :]
            pzr = pos_ref[2, pl.ds(rid, 1), :]
            lx = jnp.sum(jnp.where(selc, pxr, 0.0))
            ly = jnp.sum(jnp.where(selc, pyr, 0.0))
            lz = jnp.sum(jnp.where(selc, pzr, 0.0))
            keep = (col == i) & (gbase + i < n_samples)
            qxr = jnp.where(keep, lx, qxr)
            qyr = jnp.where(keep, ly, qyr)
            qzr = jnp.where(keep, lz, qzr)
            return (min_d, lx, ly, lz, qxr, qyr, qzr)

        start = 1 if r == 0 else 0
        min_d, lx, ly, lz, qxr, qyr, qzr = lax.fori_loop(
            start, 128, body, (min_d, lx, ly, lz, qxr, qyr, qzr))
        q_ref[0, pl.ds(r, 1), :] = qxr
        q_ref[1, pl.ds(r, 1), :] = qyr
        q_ref[2, pl.ds(r, 1), :] = qzr


def _fps(pos_grid, n_valid, n_samples, q_rows):
    return _PC(
        functools.partial(_fps_body, n_valid, n_samples, q_rows),
        out_shape=jax.ShapeDtypeStruct((3, q_rows, 128), jnp.float32),
    )(pos_grid)


def _thresh_body(r_sq, n_cand, qp_ref, pt_ref, d2_ref, t_ref):
    qx, qy, qz = qp_ref[:, 0:1], qp_ref[:, 1:2], qp_ref[:, 2:3]
    px, py, pz = pt_ref[0:1, :], pt_ref[1:2, :], pt_ref[2:3, :]
    d2 = (qx - px) ** 2 + (qy - py) ** 2 + (qz - pz) ** 2
    cid = lax.broadcasted_iota(jnp.int32, d2.shape, 1)
    d2 = jnp.where((d2 <= r_sq) & (cid < n_cand), d2, jnp.inf)
    d2_ref[...] = d2
    bits = lax.bitcast_convert_type(d2, jnp.int32)

    def body(b, r):
        cand = r | (jnp.int32(1) << (30 - b))
        cnt = jnp.sum((bits < cand[:, None]).astype(jnp.int32), axis=1)
        return jnp.where(cnt < K, cand, r)

    r = lax.fori_loop(0, 31, body, jnp.zeros((d2.shape[0],), jnp.int32))
    t_ref[...] = jnp.minimum(lax.bitcast_convert_type(r, jnp.float32),
                             jnp.float32(r_sq))


def _thresh(qpad, pos_t, r_sq, n_cand, blk=128):
    nq = qpad.shape[0]
    nc = pos_t.shape[1]
    return _PC(
        functools.partial(_thresh_body, r_sq, n_cand),
        grid=(nq // blk,),
        in_specs=[pl.BlockSpec((blk, 8), lambda i: (i, 0)),
                  pl.BlockSpec((3, nc), lambda i: (0, 0))],
        out_specs=(pl.BlockSpec((blk, nc), lambda i: (i, 0)),
                   pl.BlockSpec((blk,), lambda i: (i,))),
        out_shape=(jax.ShapeDtypeStruct((nq, nc), jnp.float32),
                   jax.ShapeDtypeStruct((nq,), jnp.float32)),
    )(qpad, pos_t)


def _sc_compact(d2, t, n_cand_pad):
    nq = d2.shape[0]
    info = plsc.get_sparse_core_info()
    nw = info.num_cores * info.num_subcores
    qpw = nq // nw
    nvec = n_cand_pad // 16
    mesh = plsc.VectorSubcoreMesh(core_axis_name="c", subcore_axis_name="s")

    @functools.partial(
        pl.kernel, mesh=mesh,
        compiler_params=pltpu.CompilerParams(needs_layout_passes=False),
        out_type=jax.ShapeDtypeStruct((nq, 128), jnp.int32),
        scratch_types=[
            pltpu.VMEM((n_cand_pad,), jnp.float32),
            pltpu.VMEM((n_cand_pad + 16,), jnp.int32),
            pltpu.VMEM((128,), jnp.int32),
            pltpu.VMEM((nq,), jnp.float32),
        ],
    )
    def k(d2_hbm, t_hbm, nbr_hbm, row_v, obuf_v, orow_v, t_v):
        wid = lax.axis_index("s") * info.num_cores + lax.axis_index("c")
        base = wid * qpw
        pltpu.sync_copy(t_hbm, t_v)
        zero16 = jnp.zeros((16,), jnp.int32)
        iota16 = lax.iota(jnp.int32, 16)

        def per_query(qi, _):
            pltpu.sync_copy(d2_hbm.at[base + qi], row_v)
            tvec = t_v[pl.ds((base + qi) // 16 * 16, 16)]
            tq = lax.gather(
                tvec, jnp.full((16, 1), (base + qi) % 16, jnp.int32),
                dimension_numbers=lax.GatherDimensionNumbers(
                    offset_dims=(), collapsed_slice_dims=(0,),
                    start_index_map=(0,)),
                slice_sizes=(1,),
                mode=lax.GatherScatterMode.PROMISE_IN_BOUNDS)

            def step(j, carry):
                o, idxv = carry
                v = row_v[pl.ds(j * 16, 16)]
                m = v <= tq
                plsc.store_compressed(obuf_v.at[pl.ds(o, 16)], idxv, mask=m)
                o = o + plsc.all_reduce_population_count(m)[0]
                return (o, idxv + 16)

            for z in range(5):
                obuf_v[pl.ds(z * 16, 16)] = zero16
            o, _ = lax.fori_loop(0, nvec, step, (jnp.int32(0), iota16),
                                 unroll=4)
            for z in range(4):
                orow_v[pl.ds(z * 16, 16)] = obuf_v[pl.ds(z * 16, 16)]
            orow_v[pl.ds(64, 16)] = jnp.minimum(o, K) + zero16
            for z in range(5, 8):
                orow_v[pl.ds(z * 16, 16)] = zero16
            pltpu.sync_copy(orow_v, nbr_hbm.at[base + qi])
            return 0

        lax.fori_loop(0, qpw, per_query, 0)

    return k(d2, t)


def _sc_gather(table, idx):
    b = idx.shape[0]
    d = table.shape[1]
    info = plsc.get_sparse_core_info()
    nw = info.num_cores * info.num_subcores
    bpw = b // nw
    chunk = bpw
    while chunk * d * 4 > 220 * 1024:
        chunk //= 2
    nchunk = bpw // chunk
    mesh = plsc.VectorSubcoreMesh(core_axis_name="c", subcore_axis_name="s")

    @functools.partial(
        pl.kernel, mesh=mesh,
        compiler_params=pltpu.CompilerParams(needs_layout_passes=False,
                                             use_tc_tiling_on_sc=False),
        out_type=jax.ShapeDtypeStruct((b, d), jnp.float32),
        scratch_types=[
            pltpu.VMEM((bpw,), jnp.int32),
            pltpu.VMEM((chunk, d), jnp.float32),
            pltpu.VMEM((chunk, d), jnp.float32),
            pltpu.SemaphoreType.DMA,
            pltpu.SemaphoreType.DMA,
        ],
    )
    def k(tab_hbm, idx_hbm, out_hbm, idx_v, rows_v0, rows_v1, sem0, sem1):
        wid = lax.axis_index("s") * info.num_cores + lax.axis_index("c")
        base = wid * bpw
        pltpu.sync_copy(idx_hbm.at[pl.ds(base, bpw)], idx_v)
        bufs, sems = (rows_v0, rows_v1), (sem0, sem1)

        def start(c):
            return pltpu.async_copy(
                tab_hbm.at[idx_v.at[pl.ds(c * chunk, chunk)]],
                bufs[c % 2], sems[c % 2])

        pend = start(0)
        for c in range(nchunk):
            pend.wait()
            if c + 1 < nchunk:
                pend = start(c + 1)
            pltpu.sync_copy(bufs[c % 2],
                            out_hbm.at[pl.ds(base + c * chunk, chunk)])

    return k(table, idx)


def _conv_body(qb, g_ref, q_ref, cnt_ref, w1_ref, b1_ref, w2_ref, b2_ref,
               w3_ref, b3_ref, o_ref):
    rows = qb * K
    qw = q_ref.shape[1]
    rep = (lax.broadcasted_iota(jnp.int32, (rows, qb), 0) // K
           == lax.broadcasted_iota(jnp.int32, (rows, qb), 1))
    qrep = jnp.dot(rep.astype(jnp.float32), q_ref[...], precision=HIGHEST)
    in1 = g_ref[..., :qw] - qrep
    h = jnp.maximum(jnp.dot(in1, w1_ref[...], precision=DEF)
                    + b1_ref[...], 0.0)
    h = jnp.maximum(jnp.dot(h, w2_ref[...], precision=DEF)
                    + b2_ref[...], 0.0)
    h = jnp.dot(h, w3_ref[...], precision=DEF) + b3_ref[...]
    ch = h.shape[1]
    h3 = h.reshape(qb, K, ch)
    slot = lax.broadcasted_iota(jnp.int32, (qb, K, 1), 1)
    valid = slot < cnt_ref[...][:, :, None]
    h3 = jnp.where(valid, h3, -1e30)
    out = jnp.max(h3, axis=1)
    o_ref[...] = jnp.where(out <= -1e29, 0.0, out)


def _conv(g, qslab, cnt, w1p, b1, w2, b2, w3, b3, qb=16):
    nq = qslab.shape[0]
    return _PC(
        functools.partial(_conv_body, qb),
        grid=(nq // qb,),
        in_specs=[pl.BlockSpec((qb * K, g.shape[1]), lambda i: (i, 0)),
                  pl.BlockSpec((qb, qslab.shape[1]), lambda i: (i, 0)),
                  pl.BlockSpec((qb, 1), lambda i: (i, 0)),
                  pl.BlockSpec(w1p.shape, lambda i: (0, 0)),
                  pl.BlockSpec((1, w1p.shape[1]), lambda i: (0, 0)),
                  pl.BlockSpec(w2.shape, lambda i: (0, 0)),
                  pl.BlockSpec((1, w2.shape[1]), lambda i: (0, 0)),
                  pl.BlockSpec(w3.shape, lambda i: (0, 0)),
                  pl.BlockSpec((1, w3.shape[1]), lambda i: (0, 0))],
        out_specs=pl.BlockSpec((qb, w3.shape[1]), lambda i: (i, 0)),
        out_shape=jax.ShapeDtypeStruct((nq, w3.shape[1]), jnp.float32),
    )(g, qslab, cnt[:, None], w1p, b1[None, :], w2, b2[None, :],
      w3, b3[None, :])


def _sa3fp3_body(n_valid, x2_ref, q2_ref, w0a_ref, w0b_ref, b0_ref, w1_ref,
                 b1_ref, w2_ref, b2_ref, fa_ref, fb_ref, fb0_ref, fc_ref,
                 fb1_ref, f3_ref):
    x2 = x2_ref[...]
    h = jnp.maximum(jnp.dot(x2, w0a_ref[...], precision=DEF)
                    + jnp.dot(q2_ref[...], w0b_ref[...],
                              precision=DEF) + b0_ref[...], 0.0)
    h = jnp.maximum(jnp.dot(h, w1_ref[...], precision=DEF)
                    + b1_ref[...], 0.0)
    h = jnp.dot(h, w2_ref[...], precision=DEF) + b2_ref[...]
    rid = lax.broadcasted_iota(jnp.int32, h.shape, 0)
    h = jnp.where(rid < n_valid, h, -jnp.inf)
    g = jnp.max(h, axis=0, keepdims=True)
    t = jnp.dot(g, fa_ref[...], precision=DEF) + fb0_ref[...]
    f = jnp.maximum(jnp.dot(x2, fb_ref[...], precision=DEF) + t, 0.0)
    f3_ref[...] = jnp.dot(f, fc_ref[...], precision=DEF) + fb1_ref[...]


def _sa3fp3(x2, q2pad, sa3, fp3):
    (w0, b0), (w1, b1), (w2, b2) = sa3
    (f0, fb0), (fc, fb1) = fp3
    fa, fb = f0[:1024, :], f0[1024:, :]
    w0a = w0[:256, :]
    w0b = jnp.zeros((8, 256), jnp.float32).at[:3, :].set(w0[256:, :])
    nq = x2.shape[0]
    return _PC(
        functools.partial(_sa3fp3_body, N2),
        out_shape=jax.ShapeDtypeStruct((nq, fc.shape[1]), jnp.float32),
    )(x2, q2pad, w0a, w0b, b0[None, :], w1, b1[None, :], w2, b2[None, :],
      fa, fb, fb0[None, :], fc, fb1[None, :])


def _knn3(tp_ref, st_ref, n_src):
    qx, qy, qz = tp_ref[:, 0:1], tp_ref[:, 1:2], tp_ref[:, 2:3]
    px, py, pz = st_ref[0:1, :], st_ref[1:2, :], st_ref[2:3, :]
    d2 = (qx - px) ** 2 + (qy - py) ** 2 + (qz - pz) ** 2
    cid = lax.broadcasted_iota(jnp.int32, d2.shape, 1)
    d2 = jnp.where(cid < n_src, d2, jnp.inf)
    wmat = jnp.zeros(d2.shape, jnp.float32)
    sumw = jnp.zeros((d2.shape[0], 1), jnp.float32)
    for _ in range(3):
        m = jnp.min(d2, axis=1, keepdims=True)
        idx = jnp.min(jnp.where(d2 == m, cid, BIGI), axis=1, keepdims=True)
        w = 1.0 / jnp.maximum(m, 1e-16)
        wmat = wmat + jnp.where(cid == idx, w, 0.0)
        sumw = sumw + w
        d2 = jnp.where(cid == idx, jnp.inf, d2)
    return wmat, sumw


def _knn_fp2_body(n_src, tp_ref, st_ref, f_ref, x1_ref, v0a_ref, v0b_ref,
                  c0_ref, v1_ref, c1_ref, o_ref):
    wmat, sumw = _knn3(tp_ref, st_ref, n_src)
    xi = jnp.dot(wmat, f_ref[...], precision=HIGHEST) / sumw
    h = jnp.maximum(jnp.dot(xi, v0a_ref[...], precision=DEF)
                    + jnp.dot(x1_ref[...], v0b_ref[...], precision=DEF)
                    + c0_ref[...], 0.0)
    o_ref[...] = jnp.dot(h, v1_ref[...], precision=DEF) + c1_ref[...]


def _knn_fp2(tpad, s_t, f3, x1, fp2, n_src, blk=128):
    (v0, c0), (v1, c1) = fp2
    v0a, v0b = v0[:256, :], v0[256:, :]
    nt = tpad.shape[0]
    ns = s_t.shape[1]
    return _PC(
        functools.partial(_knn_fp2_body, n_src),
        grid=(nt // blk,),
        in_specs=[pl.BlockSpec((blk, 8), lambda i: (i, 0)),
                  pl.BlockSpec((3, ns), lambda i: (0, 0)),
                  pl.BlockSpec(f3.shape, lambda i: (0, 0)),
                  pl.BlockSpec((blk, x1.shape[1]), lambda i: (i, 0)),
                  pl.BlockSpec(v0a.shape, lambda i: (0, 0)),
                  pl.BlockSpec(v0b.shape, lambda i: (0, 0)),
                  pl.BlockSpec((1, v0.shape[1]), lambda i: (0, 0)),
                  pl.BlockSpec(v1.shape, lambda i: (0, 0)),
                  pl.BlockSpec((1, v1.shape[1]), lambda i: (0, 0))],
        out_specs=pl.BlockSpec((blk, v1.shape[1]), lambda i: (i, 0)),
        out_shape=jax.ShapeDtypeStruct((nt, v1.shape[1]), jnp.float32),
    )(tpad, s_t, f3, x1, v0a, v0b, c0[None, :], v1, c1[None, :])


def _knn_fp1_head_body(n_src, tp_ref, xf_ref, st_ref, f_ref, u0a_ref,
                       u0b_ref, d0_ref, u1_ref, d1_ref, u2_ref, d2r_ref,
                       h0_ref, e0_ref, h1_ref, e1_ref, h2_ref, e2_ref,
                       o_ref):
    wmat, sumw = _knn3(tp_ref, st_ref, n_src)
    xi = jnp.dot(wmat, f_ref[...], precision=HIGHEST) / sumw
    h = jnp.maximum(jnp.dot(xi, u0a_ref[...], precision=DEF)
                    + jnp.dot(xf_ref[...], u0b_ref[...], precision=DEF)
                    + d0_ref[...], 0.0)
    h = jnp.maximum(jnp.dot(h, u1_ref[...], precision=DEF)
                    + d1_ref[...], 0.0)
    h = jnp.dot(h, u2_ref[...], precision=DEF) + d2r_ref[...]
    h = jnp.maximum(jnp.dot(h, h0_ref[...], precision=DEF)
                    + e0_ref[...], 0.0)
    h = jnp.maximum(jnp.dot(h, h1_ref[...], precision=DEF)
                    + e1_ref[...], 0.0)
    o_ref[...] = jnp.dot(h, h2_ref[...], precision=DEF) + e2_ref[...]


def _knn_fp1_head(ppad, xfpad, s_t, f2, fp1, head, n_src, blk=128):
    (u0, d0), (u1, d1), (u2, d2) = fp1
    (h0, e0), (h1, e1), (h2, e2) = head
    u0a = u0[:128, :]
    u0b = jnp.zeros((8, 128), jnp.float32).at[:3, :].set(u0[128:, :])
    h2p = jnp.zeros((128, 128), jnp.float32).at[:, :13].set(h2)
    e2p = jnp.zeros((128,), jnp.float32).at[:13].set(e2)
    nt = ppad.shape[0]
    ns = s_t.shape[1]
    args = (ppad, xfpad, s_t, f2, u0a, u0b, d0[None, :], u1, d1[None, :],
            u2, d2[None, :], h0, e0[None, :], h1, e1[None, :], h2p,
            e2p[None, :])
    specs = [pl.BlockSpec((blk, 8), lambda i: (i, 0)),
             pl.BlockSpec((blk, 8), lambda i: (i, 0)),
             pl.BlockSpec((3, ns), lambda i: (0, 0)),
             pl.BlockSpec(f2.shape, lambda i: (0, 0))]
    for a in args[4:]:
        specs.append(pl.BlockSpec(a.shape, lambda i: (0, 0)))
    return _PC(
        functools.partial(_knn_fp1_head_body, n_src),
        grid=(nt // blk,),
        in_specs=specs,
        out_specs=pl.BlockSpec((blk, 128), lambda i: (i, 0)),
        out_shape=jax.ShapeDtypeStruct((nt, 128), jnp.float32),
    )(*args)


def _pad_rows(a, n):
    return jnp.zeros((n, a.shape[1]), a.dtype).at[:a.shape[0], :].set(a)


def kernel(x, pos, batch, params):
    del batch
    f32 = jnp.float32
    pos_grid = pos.T.reshape(3, 64, 128)

    q1g = _fps(pos_grid, N, N1, N1P // 128)
    q2g = _fps(q1g, N1, N2, N2P // 128)
    q1pad = jnp.zeros((N1P, 8), f32).at[:, :3].set(q1g.reshape(3, N1P).T)
    q2pad = jnp.zeros((N2P, 8), f32).at[:, :3].set(q2g.reshape(3, N2P).T)
    pos_t = pos.T
    q1_t = q1g.reshape(3, N1P)
    q2_t = q2g.reshape(3, N2P)

    sa1 = params['sa1']
    w1 = sa1[0][0]
    t1tab = (jnp.zeros((N, 16), f32)
             .at[:, 0:3].set(x).at[:, 8:11].set(pos))
    q1slab = jnp.zeros((N1P, 16), f32).at[:, 8:11].set(
        q1g.reshape(3, N1P).T)
    w1p = (jnp.zeros((16, 64), f32)
           .at[0:3, :].set(w1[0:3, :]).at[8:11, :].set(w1[3:6, :]))
    d2_1, t1 = _thresh(q1pad, pos_t, R1SQ, N)
    pk1 = _sc_compact(d2_1, t1, N)
    nbr1, cnt1 = pk1[:, :K], pk1[:, K]
    g1 = _sc_gather(t1tab, nbr1.reshape(-1))
    x1 = _conv(g1, q1slab, cnt1, w1p, sa1[0][1], sa1[1][0], sa1[1][1],
               sa1[2][0], sa1[2][1])

    sa2 = params['sa2']
    w2 = sa2[0][0]
    t2tab = (jnp.zeros((N1P, 136), f32)
             .at[:, 0:128].set(x1).at[:, 128:131].set(q1g.reshape(3, N1P).T))
    q2slab = jnp.zeros((N2P, 136), f32).at[:, 128:131].set(
        q2g.reshape(3, N2P).T)
    w2p = jnp.zeros((136, 128), f32).at[0:131, :].set(w2)
    d2_2, t2 = _thresh(q2pad, q1_t, R2SQ, N1)
    pk2 = _sc_compact(d2_2, t2, N1P)
    nbr2, cnt2 = pk2[:, :K], pk2[:, K]
    g2 = _sc_gather(t2tab, nbr2.reshape(-1))
    x2 = _conv(g2, q2slab, cnt2, w2p, sa2[0][1], sa2[1][0], sa2[1][1],
               sa2[2][0], sa2[2][1])

    f3 = _sa3fp3(x2, q2pad, params['sa3'], params['fp3'])

    f2 = _knn_fp2(q1pad, q2_t, f3, x1, params['fp2'], N2)

    ppad = jnp.zeros((N, 8), f32).at[:, :3].set(pos)
    xfpad = jnp.zeros((N, 8), f32).at[:, :3].set(x)
    out = _knn_fp1_head(ppad, xfpad, q1_t, f2, params['fp1'],
                        params['head'], N1)
    return out[:, :13]


def _prep2_body(x1_ref, q1_ref, q2_ref, wx_ref, wp_ref, b_ref,
                a_ref, c_ref):
    a_ref[...] = (jnp.dot(x1_ref[...], wx_ref[...], precision=HIGHEST)
                  + jnp.dot(q1_ref[...], wp_ref[...], precision=HIGHEST))
    c_ref[...] = (b_ref[...]
                  - jnp.dot(q2_ref[...], wp_ref[...], precision=HIGHEST))


def _prep2(x1, q1pad, q2pad, wx, wp_pad, b):
    return _PC(
        _prep2_body,
        out_shape=(jax.ShapeDtypeStruct((N1P, wx.shape[1]), jnp.float32),
                   jax.ShapeDtypeStruct((N2P, wx.shape[1]), jnp.float32)),
    )(x1, q1pad, q2pad, wx, wp_pad, b[None, :])
